# baseline (speedup 1.0000x reference)
import jax
import jax.numpy as jnp
from jax.experimental import pallas as pl
from jax.experimental.pallas import tpu as pltpu
from jax.experimental.pallas import tpu_sc as plsc


def _overlap_body(a_ref, g_ref, o_ref):
    g = g_ref[...]
    c1 = jnp.dot(a_ref[...], g, preferred_element_type=jnp.float32)
    c2 = jnp.dot(a_ref[...], c1, preferred_element_type=jnp.float32)
    o_ref[...] = g + c1 + c2


def _overlap(a, g):
    return pl.pallas_call(
        _overlap_body,
        out_shape=jax.ShapeDtypeStruct(g.shape, jnp.float32),
    )(a, g)


def _mm_body(a_ref, x_ref, o_ref):
    o_ref[...] = jnp.dot(a_ref[...], x_ref[...],
                         preferred_element_type=jnp.float32)


def _rowmm(a, x, bm):
    m, k = a.shape
    n = x.shape[1]
    return pl.pallas_call(
        _mm_body,
        grid=(m // bm,),
        in_specs=[pl.BlockSpec((bm, k), lambda i: (i, 0)),
                  pl.BlockSpec((k, n), lambda i: (0, 0))],
        out_specs=pl.BlockSpec((bm, n), lambda i: (i, 0)),
        out_shape=jax.ShapeDtypeStruct((m, n), jnp.float32),
    )(a, x)


def _msg_body(uh_ref, ih_ref, u_ref, it_ref, ge_ref, w_ref, b_ref, o_ref):
    d = ge_ref.shape[1]
    um = jnp.dot(uh_ref[...], u_ref[...], preferred_element_type=jnp.float32)
    im = jnp.dot(ih_ref[...], it_ref[...], preferred_element_type=jnp.float32)
    ig = im * ge_ref[...]
    w = w_ref[...]
    o_ref[...] = (jnp.dot(um, w[0:d], preferred_element_type=jnp.float32)
                  + jnp.dot(im, w[d:2 * d], preferred_element_type=jnp.float32)
                  + jnp.dot(ig, w[2 * d:3 * d], preferred_element_type=jnp.float32)
                  + b_ref[...])


def _msg(uh, ih, u, it, ge, w, b, bm):
    g, nu = uh.shape
    ni = ih.shape[1]
    d = ge.shape[1]
    return pl.pallas_call(
        _msg_body,
        grid=(g // bm,),
        in_specs=[pl.BlockSpec((bm, nu), lambda i: (i, 0)),
                  pl.BlockSpec((bm, ni), lambda i: (i, 0)),
                  pl.BlockSpec((nu, d), lambda i: (0, 0)),
                  pl.BlockSpec((ni, d), lambda i: (0, 0)),
                  pl.BlockSpec((bm, d), lambda i: (i, 0)),
                  pl.BlockSpec((3 * d, d), lambda i: (0, 0)),
                  pl.BlockSpec((1, d), lambda i: (0, 0))],
        out_specs=pl.BlockSpec((bm, d), lambda i: (i, 0)),
        out_shape=jax.ShapeDtypeStruct((g, d), jnp.float32),
    )(uh, ih, u, it, ge, w, b)


def _fuse_body(ge_ref, m1_ref, m2_ref, l0_ref, l1_ref, l2_ref,
               it0_ref, n1_ref, n2_ref,
               wov_ref, bov_ref, why_ref, bhy_ref, wlg_ref, blg_ref,
               gout_ref, iout_ref):
    ge = ge_ref[...]
    he = ge + m1_ref[...] + m2_ref[...]
    lg = (l0_ref[...] + l1_ref[...] + l2_ref[...]) * (1.0 / 3.0)
    co = jax.nn.sigmoid(
        jnp.dot(ge, wov_ref[...], preferred_element_type=jnp.float32)
        + bov_ref[...])
    ch = jax.nn.sigmoid(
        jnp.dot(he, why_ref[...], preferred_element_type=jnp.float32)
        + bhy_ref[...])
    cl = jax.nn.sigmoid(
        jnp.dot(lg, wlg_ref[...], preferred_element_type=jnp.float32)
        + blg_ref[...])
    g_val = co * ge + ch * he + cl * lg
    i_val = it0_ref[...] + n1_ref[...] + n2_ref[...]
    gout_ref[...] = jnp.concatenate([g_val, jnp.zeros_like(g_val)], axis=1)
    iout_ref[...] = jnp.concatenate([i_val, jnp.zeros_like(i_val)], axis=1)


def _fuse(ge, m1, m2, l0, l1, l2, it0, n1, n2, wov, bov, why, bhy, wlg, blg):
    g, d = ge.shape
    i = it0.shape[0]
    return pl.pallas_call(
        _fuse_body,
        out_shape=(jax.ShapeDtypeStruct((g, 2 * d), jnp.float32),
                   jax.ShapeDtypeStruct((i, 2 * d), jnp.float32)),
    )(ge, m1, m2, l0, l1, l2, it0, n1, n2,
      wov, bov.reshape(1, 1), why, bhy.reshape(1, 1), wlg, blg.reshape(1, 1))


def _gather_pair(gui, iemb, gidx, iidx):
    b = gidx.shape[0]
    d = gui.shape[1]
    w = 128
    mesh = plsc.VectorSubcoreMesh(core_axis_name="c", subcore_axis_name="s")
    gi2 = gidx.reshape(1, b)
    ii2 = iidx.reshape(1, b)

    @pl.kernel(out_type=(jax.ShapeDtypeStruct((b, d), jnp.float32),
                         jax.ShapeDtypeStruct((b, d), jnp.float32)),
               mesh=mesh)
    def k(gui_hbm, iemb_hbm, gi_hbm, ii_hbm, og_hbm, oi_hbm):
        def body(gi_vmem, ii_vmem, og_vmem, oi_vmem):
            pltpu.sync_copy(gui_hbm.at[gi_vmem.at[0]], og_vmem)
            pltpu.sync_copy(iemb_hbm.at[ii_vmem.at[0]], oi_vmem)

        pltpu.emit_pipeline(
            body,
            grid=(b // w,),
            in_specs=[pl.BlockSpec((1, w), lambda i: (0, i)),
                      pl.BlockSpec((1, w), lambda i: (0, i))],
            out_specs=[pl.BlockSpec((w, d), lambda i: (i, 0)),
                       pl.BlockSpec((w, d), lambda i: (i, 0))],
            core_axis_name=("c", "s"),
            dimension_semantics=(pltpu.PARALLEL,),
        )(gi_hbm, ii_hbm, og_hbm, oi_hbm)

    return k(gui, iemb, gi2, ii2)


def _dot_body(g_ref, i_ref, o_ref):
    o_ref[...] = jnp.sum(g_ref[...] * i_ref[...], axis=1, keepdims=True)


def _dot(gs, isel, bm):
    b, d = gs.shape
    out = pl.pallas_call(
        _dot_body,
        grid=(b // bm,),
        in_specs=[pl.BlockSpec((bm, d), lambda i: (i, 0)),
                  pl.BlockSpec((bm, d), lambda i: (i, 0))],
        out_specs=pl.BlockSpec((bm, 1), lambda i: (i, 0)),
        out_shape=jax.ShapeDtypeStruct((b, 1), jnp.float32),
    )(gs, isel)
    return out.reshape(b)


def kernel(user_table, item_table, group_table, user_hyper, item_hyper,
           full_hyper, overlap_graph, lgcn_graph, W_agg, b_agg,
           W_ov, b_ov, W_hy, b_hy, W_lg, b_lg,
           group_inputs, item_inputs):
    nu, d = user_table.shape
    ni = item_table.shape[0]
    g = group_table.shape[0]
    nlayers = W_agg.shape[0]
    nlg = lgcn_graph.shape[0]

    group_emb = _overlap(overlap_graph, group_table)

    u, it = user_table, item_table
    msgs = []
    norms_i = []
    for l in range(nlayers):
        msg = _msg(user_hyper, item_hyper, u, it, group_emb,
                   W_agg[l], b_agg[l].reshape(1, d), bm=200)
        norm = _rowmm(full_hyper, msg, bm=600)
        u = norm[:nu]
        it = norm[nu:]
        msgs.append(msg)
        norms_i.append(it)

    e0 = jnp.concatenate([group_table, item_table[:nlg - g]], axis=0)
    c1 = _rowmm(lgcn_graph, e0, bm=200)
    c2 = _rowmm(lgcn_graph, c1, bm=200)

    group_ui, i_emb = _fuse(group_emb, msgs[0], msgs[1],
                            group_table, c1[:g], c2[:g],
                            item_table, norms_i[0], norms_i[1],
                            W_ov, b_ov, W_hy, b_hy, W_lg, b_lg)

    g_sel, i_sel = _gather_pair(group_ui, i_emb, group_inputs, item_inputs)
    return _dot(g_sel, i_sel, bm=2048)


# speedup vs baseline: 1.1305x; 1.1305x over previous
import jax
import jax.numpy as jnp
from jax.experimental import pallas as pl
from jax.experimental.pallas import tpu as pltpu
from jax.experimental.pallas import tpu_sc as plsc


def _overlap_body(a_ref, g_ref, o_ref):
    g = g_ref[...]
    c1 = jnp.dot(a_ref[...], g, preferred_element_type=jnp.float32)
    c2 = jnp.dot(a_ref[...], c1, preferred_element_type=jnp.float32)
    o_ref[...] = g + c1 + c2


def _overlap(a, g):
    return pl.pallas_call(
        _overlap_body,
        out_shape=jax.ShapeDtypeStruct(g.shape, jnp.float32),
    )(a, g)


def _mm_body(a_ref, x_ref, o_ref):
    o_ref[...] = jnp.dot(a_ref[...], x_ref[...],
                         preferred_element_type=jnp.float32)


def _rowmm(a, x, bm, row0=0, mrows=None):
    m, k = a.shape
    n = x.shape[1]
    if mrows is None:
        mrows = m
    off = row0 // bm
    return pl.pallas_call(
        _mm_body,
        grid=(mrows // bm,),
        in_specs=[pl.BlockSpec((bm, k), lambda i: (i + off, 0)),
                  pl.BlockSpec((k, n), lambda i: (0, 0))],
        out_specs=pl.BlockSpec((bm, n), lambda i: (i, 0)),
        out_shape=jax.ShapeDtypeStruct((mrows, n), jnp.float32),
    )(a, x)


def _msg_body(uh_ref, ih_ref, u_ref, it_ref, ge_ref, w_ref, b_ref, o_ref):
    d = ge_ref.shape[1]
    um = jnp.dot(uh_ref[...], u_ref[...], preferred_element_type=jnp.float32)
    im = jnp.dot(ih_ref[...], it_ref[...], preferred_element_type=jnp.float32)
    ig = im * ge_ref[...]
    w = w_ref[...]
    o_ref[...] = (jnp.dot(um, w[0:d], preferred_element_type=jnp.float32)
                  + jnp.dot(im, w[d:2 * d], preferred_element_type=jnp.float32)
                  + jnp.dot(ig, w[2 * d:3 * d], preferred_element_type=jnp.float32)
                  + b_ref[...])


def _msg(uh, ih, u, it, ge, w, b, bm):
    g, nu = uh.shape
    ni = ih.shape[1]
    d = ge.shape[1]
    return pl.pallas_call(
        _msg_body,
        grid=(g // bm,),
        in_specs=[pl.BlockSpec((bm, nu), lambda i: (i, 0)),
                  pl.BlockSpec((bm, ni), lambda i: (i, 0)),
                  pl.BlockSpec((nu, d), lambda i: (0, 0)),
                  pl.BlockSpec((ni, d), lambda i: (0, 0)),
                  pl.BlockSpec((bm, d), lambda i: (i, 0)),
                  pl.BlockSpec((3 * d, d), lambda i: (0, 0)),
                  pl.BlockSpec((1, d), lambda i: (0, 0))],
        out_specs=pl.BlockSpec((bm, d), lambda i: (i, 0)),
        out_shape=jax.ShapeDtypeStruct((g, d), jnp.float32),
    )(uh, ih, u, it, ge, w, b)


def _isum_body(it0_ref, n1_ref, n2_ref, o_ref):
    v = it0_ref[...] + n1_ref[...] + n2_ref[...]
    o_ref[...] = jnp.concatenate([v, jnp.zeros_like(v)], axis=1)


def _isum(it0, n1, n2):
    i, d = it0.shape
    return pl.pallas_call(
        _isum_body,
        out_shape=jax.ShapeDtypeStruct((i, 2 * d), jnp.float32),
    )(it0, n1, n2)


def _fuse_body(ge_ref, m1_ref, m2_ref, l0_ref, l1_ref, l2_ref,
               wov_ref, bov_ref, why_ref, bhy_ref, wlg_ref, blg_ref,
               gout_ref):
    ge = ge_ref[...]
    he = ge + m1_ref[...] + m2_ref[...]
    lg = (l0_ref[...] + l1_ref[...] + l2_ref[...]) * (1.0 / 3.0)
    co = jax.nn.sigmoid(
        jnp.dot(ge, wov_ref[...], preferred_element_type=jnp.float32)
        + bov_ref[...])
    ch = jax.nn.sigmoid(
        jnp.dot(he, why_ref[...], preferred_element_type=jnp.float32)
        + bhy_ref[...])
    cl = jax.nn.sigmoid(
        jnp.dot(lg, wlg_ref[...], preferred_element_type=jnp.float32)
        + blg_ref[...])
    g_val = co * ge + ch * he + cl * lg
    gout_ref[...] = jnp.concatenate([g_val, jnp.zeros_like(g_val)], axis=1)


def _fuse(ge, m1, m2, l0, l1, l2, wov, bov, why, bhy, wlg, blg):
    g, d = ge.shape
    return pl.pallas_call(
        _fuse_body,
        out_shape=jax.ShapeDtypeStruct((g, 2 * d), jnp.float32),
    )(ge, m1, m2, l0, l1, l2,
      wov, bov.reshape(1, 1), why, bhy.reshape(1, 1), wlg, blg.reshape(1, 1))


def _sc_gather(table, idx):
    b = idx.shape[0]
    d = table.shape[1]
    w = 128
    mesh = plsc.VectorSubcoreMesh(core_axis_name="c", subcore_axis_name="s")
    idx2 = idx.reshape(1, b)

    @pl.kernel(out_type=jax.ShapeDtypeStruct((b, d), jnp.float32),
               mesh=mesh)
    def k(tab_hbm, i_hbm, o_hbm):
        def body(i_vmem, o_vmem):
            pltpu.sync_copy(tab_hbm.at[i_vmem.at[0]], o_vmem)

        pltpu.emit_pipeline(
            body,
            grid=(b // w,),
            in_specs=[pl.BlockSpec((1, w), lambda i: (0, i))],
            out_specs=[pl.BlockSpec((w, d), lambda i: (i, 0))],
            core_axis_name=("c", "s"),
            dimension_semantics=(pltpu.PARALLEL,),
        )(i_hbm, o_hbm)

    return k(table, idx2)


def _dot_body(g_ref, i_ref, o_ref):
    o_ref[...] = jnp.sum(g_ref[...] * i_ref[...], axis=1, keepdims=True)


def _dot(gs, isel, bm):
    b, d = gs.shape
    out = pl.pallas_call(
        _dot_body,
        grid=(b // bm,),
        in_specs=[pl.BlockSpec((bm, d), lambda i: (i, 0)),
                  pl.BlockSpec((bm, d), lambda i: (i, 0))],
        out_specs=pl.BlockSpec((bm, 1), lambda i: (i, 0)),
        out_shape=jax.ShapeDtypeStruct((b, 1), jnp.float32),
    )(gs, isel)
    return out.reshape(b)


def kernel(user_table, item_table, group_table, user_hyper, item_hyper,
           full_hyper, overlap_graph, lgcn_graph, W_agg, b_agg,
           W_ov, b_ov, W_hy, b_hy, W_lg, b_lg,
           group_inputs, item_inputs):
    nu, d = user_table.shape
    ni = item_table.shape[0]
    g = group_table.shape[0]
    nlayers = W_agg.shape[0]
    nlg = lgcn_graph.shape[0]

    group_emb = _overlap(overlap_graph, group_table)

    msg1 = _msg(user_hyper, item_hyper, user_table, item_table, group_emb,
                W_agg[0], b_agg[0].reshape(1, d), bm=200)
    norm1 = _rowmm(full_hyper, msg1, bm=600)
    u1 = norm1[:nu]
    it1 = norm1[nu:]

    msg2 = _msg(user_hyper, item_hyper, u1, it1, group_emb,
                W_agg[1], b_agg[1].reshape(1, d), bm=200)
    it2 = _rowmm(full_hyper, msg2, bm=200, row0=nu, mrows=ni)

    i_emb = _isum(item_table, it1, it2)
    i_sel = _sc_gather(i_emb, item_inputs)

    e0 = jnp.concatenate([group_table, item_table[:nlg - g]], axis=0)
    c1 = _rowmm(lgcn_graph, e0, bm=200)
    c2g = _rowmm(lgcn_graph, c1, bm=200, mrows=g)

    group_ui = _fuse(group_emb, msg1, msg2, group_table, c1[:g], c2g,
                     W_ov, b_ov, W_hy, b_hy, W_lg, b_lg)
    g_sel = _sc_gather(group_ui, group_inputs)

    return _dot(g_sel, i_sel, bm=2048)


# speedup vs baseline: 1.1410x; 1.0093x over previous
import jax
import jax.numpy as jnp
from jax.experimental import pallas as pl
from jax.experimental.pallas import tpu as pltpu
from jax.experimental.pallas import tpu_sc as plsc


def _start(src, dst, sem):
    pltpu.make_async_copy(src, dst, sem).start()


def _wait(src, dst, sem):
    pltpu.make_async_copy(src, dst, sem).wait()


def _mk1(u0, it0, g0, uh, ih, fh, a, wagg, bagg):
    nu, d = u0.shape
    ni = it0.shape[0]
    g = g0.shape[0]
    bm_g = 200
    bm_f1 = 600
    bm_f2 = 200
    f2_off = nu // bm_f2

    def body(u0_hbm, it0_hbm, g0_hbm, uh_hbm, ih_hbm, fh_hbm, a_hbm,
             wagg_hbm, bagg_hbm,
             oge_hbm, omsg1_hbm, omsg2_hbm, onorm1_hbm, oiemb_hbm,
             s_a, s_u, s_it, s_g, s_ge, s_msg, s_w, s_b, sem):
        first = ((u0_hbm, s_u), (it0_hbm, s_it), (g0_hbm, s_g),
                 (a_hbm, s_a), (wagg_hbm.at[0], s_w), (bagg_hbm.at[0:1], s_b))
        for src, dst in first:
            _start(src, dst, sem)
        for src, dst in first:
            _wait(src, dst, sem)

        gv = s_g[...]
        c1 = jnp.dot(s_a[...], gv, preferred_element_type=jnp.float32)
        c2 = jnp.dot(s_a[...], c1, preferred_element_type=jnp.float32)
        s_ge[...] = gv + c1 + c2
        _start(s_ge, oge_hbm, sem)
        _wait(s_ge, oge_hbm, sem)

        def msg_body(uh_ref, ih_ref, ge_ref, o_ref):
            um = jnp.dot(uh_ref[...], s_u[...],
                         preferred_element_type=jnp.float32)
            im = jnp.dot(ih_ref[...], s_it[...],
                         preferred_element_type=jnp.float32)
            ig = im * ge_ref[...]
            w = s_w[...]
            o_ref[...] = (jnp.dot(um, w[0:d], preferred_element_type=jnp.float32)
                          + jnp.dot(im, w[d:2 * d],
                                    preferred_element_type=jnp.float32)
                          + jnp.dot(ig, w[2 * d:3 * d],
                                    preferred_element_type=jnp.float32)
                          + s_b[...])

        def run_msg(omsg_hbm):
            pltpu.emit_pipeline(
                msg_body,
                grid=(g // bm_g,),
                in_specs=[pl.BlockSpec((bm_g, nu), lambda i: (i, 0)),
                          pl.BlockSpec((bm_g, ni), lambda i: (i, 0)),
                          pl.BlockSpec((bm_g, d), lambda i: (i, 0))],
                out_specs=[pl.BlockSpec((bm_g, d), lambda i: (i, 0))],
            )(uh_hbm, ih_hbm, oge_hbm, omsg_hbm)

        run_msg(omsg1_hbm)
        _start(omsg1_hbm, s_msg, sem)
        _wait(omsg1_hbm, s_msg, sem)

        def prop_body(fh_ref, o_ref):
            o_ref[...] = jnp.dot(fh_ref[...], s_msg[...],
                                 preferred_element_type=jnp.float32)

        pltpu.emit_pipeline(
            prop_body,
            grid=((nu + ni) // bm_f1,),
            in_specs=[pl.BlockSpec((bm_f1, g), lambda i: (i, 0))],
            out_specs=[pl.BlockSpec((bm_f1, d), lambda i: (i, 0))],
        )(fh_hbm, onorm1_hbm)

        second = ((onorm1_hbm.at[0:nu], s_u), (onorm1_hbm.at[nu:nu + ni], s_it),
                  (wagg_hbm.at[1], s_w), (bagg_hbm.at[1:2], s_b))
        for src, dst in second:
            _start(src, dst, sem)
        for src, dst in second:
            _wait(src, dst, sem)

        run_msg(omsg2_hbm)
        _start(omsg2_hbm, s_msg, sem)
        _wait(omsg2_hbm, s_msg, sem)

        def iemb_body(fh_ref, it0_ref, n1_ref, o_ref):
            it2 = jnp.dot(fh_ref[...], s_msg[...],
                          preferred_element_type=jnp.float32)
            v = it0_ref[...] + n1_ref[...] + it2
            o_ref[...] = jnp.concatenate([v, jnp.zeros_like(v)], axis=1)

        pltpu.emit_pipeline(
            iemb_body,
            grid=(ni // bm_f2,),
            in_specs=[pl.BlockSpec((bm_f2, g), lambda i: (i + f2_off, 0)),
                      pl.BlockSpec((bm_f2, d), lambda i: (i, 0)),
                      pl.BlockSpec((bm_f2, d), lambda i: (i + f2_off, 0))],
            out_specs=[pl.BlockSpec((bm_f2, 2 * d), lambda i: (i, 0))],
        )(fh_hbm, it0_hbm, onorm1_hbm, oiemb_hbm)

    anyspec = pl.BlockSpec(memory_space=pltpu.MemorySpace.HBM)
    f32 = jnp.float32
    out = pl.pallas_call(
        body,
        in_specs=[anyspec] * 9,
        out_specs=[anyspec] * 5,
        out_shape=(jax.ShapeDtypeStruct((g, d), f32),
                   jax.ShapeDtypeStruct((g, d), f32),
                   jax.ShapeDtypeStruct((g, d), f32),
                   jax.ShapeDtypeStruct((nu + ni, d), f32),
                   jax.ShapeDtypeStruct((ni, 2 * d), f32)),
        scratch_shapes=[pltpu.VMEM((g, g), f32),
                        pltpu.VMEM((nu, d), f32),
                        pltpu.VMEM((ni, d), f32),
                        pltpu.VMEM((g, d), f32),
                        pltpu.VMEM((g, d), f32),
                        pltpu.VMEM((g, d), f32),
                        pltpu.VMEM((3 * d, d), f32),
                        pltpu.VMEM((1, d), f32),
                        pltpu.SemaphoreType.DMA],
    )(u0, it0, g0, uh, ih, fh, a, wagg, bagg)
    return out


def _mk2(lg, g0, it0, ge, m1, m2, wov, why, wlg, bov, bhy, blg):
    g, d = g0.shape
    nlg = lg.shape[0]
    bm = 200

    def body(lg_hbm, g0_hbm, it0_hbm, ge_hbm, m1_hbm, m2_hbm,
             wov_hbm, why_hbm, wlg_hbm, bov_hbm, bhy_hbm, blg_hbm,
             oc1_hbm, ogui_hbm,
             s_e0, s_c1, s_wov, s_why, s_wlg, s_bov, s_bhy, s_blg, sem):
        first = ((g0_hbm, s_e0.at[0:g]), (it0_hbm.at[0:nlg - g], s_e0.at[g:nlg]),
                 (wov_hbm, s_wov), (why_hbm, s_why), (wlg_hbm, s_wlg),
                 (bov_hbm, s_bov), (bhy_hbm, s_bhy), (blg_hbm, s_blg))
        for src, dst in first:
            _start(src, dst, sem)
        for src, dst in first:
            _wait(src, dst, sem)

        def lg1_body(lg_ref, o_ref):
            o_ref[...] = jnp.dot(lg_ref[...], s_e0[...],
                                 preferred_element_type=jnp.float32)

        pltpu.emit_pipeline(
            lg1_body,
            grid=(nlg // bm,),
            in_specs=[pl.BlockSpec((bm, nlg), lambda i: (i, 0))],
            out_specs=[pl.BlockSpec((bm, d), lambda i: (i, 0))],
        )(lg_hbm, oc1_hbm)

        _start(oc1_hbm, s_c1, sem)
        _wait(oc1_hbm, s_c1, sem)

        def fuse_body(lg_ref, ge_ref, m1_ref, m2_ref, g0_ref, c1_ref, o_ref):
            c2 = jnp.dot(lg_ref[...], s_c1[...],
                         preferred_element_type=jnp.float32)
            ge_v = ge_ref[...]
            he = ge_v + m1_ref[...] + m2_ref[...]
            lgx = (g0_ref[...] + c1_ref[...] + c2) * (1.0 / 3.0)
            co = jax.nn.sigmoid(
                jnp.dot(ge_v, s_wov[...], preferred_element_type=jnp.float32)
                + s_bov[...])
            ch = jax.nn.sigmoid(
                jnp.dot(he, s_why[...], preferred_element_type=jnp.float32)
                + s_bhy[...])
            cl = jax.nn.sigmoid(
                jnp.dot(lgx, s_wlg[...], preferred_element_type=jnp.float32)
                + s_blg[...])
            v = co * ge_v + ch * he + cl * lgx
            o_ref[...] = jnp.concatenate([v, jnp.zeros_like(v)], axis=1)

        pltpu.emit_pipeline(
            fuse_body,
            grid=(g // bm,),
            in_specs=[pl.BlockSpec((bm, nlg), lambda i: (i, 0)),
                      pl.BlockSpec((bm, d), lambda i: (i, 0)),
                      pl.BlockSpec((bm, d), lambda i: (i, 0)),
                      pl.BlockSpec((bm, d), lambda i: (i, 0)),
                      pl.BlockSpec((bm, d), lambda i: (i, 0)),
                      pl.BlockSpec((bm, d), lambda i: (i, 0))],
            out_specs=[pl.BlockSpec((bm, 2 * d), lambda i: (i, 0))],
        )(lg_hbm, ge_hbm, m1_hbm, m2_hbm, g0_hbm, oc1_hbm, ogui_hbm)

    anyspec = pl.BlockSpec(memory_space=pltpu.MemorySpace.HBM)
    f32 = jnp.float32
    out = pl.pallas_call(
        body,
        in_specs=[anyspec] * 12,
        out_specs=[anyspec] * 2,
        out_shape=(jax.ShapeDtypeStruct((nlg, d), f32),
                   jax.ShapeDtypeStruct((g, 2 * d), f32)),
        scratch_shapes=[pltpu.VMEM((nlg, d), f32),
                        pltpu.VMEM((nlg, d), f32),
                        pltpu.VMEM((d, 1), f32),
                        pltpu.VMEM((d, 1), f32),
                        pltpu.VMEM((d, 1), f32),
                        pltpu.VMEM((1, 1), f32),
                        pltpu.VMEM((1, 1), f32),
                        pltpu.VMEM((1, 1), f32),
                        pltpu.SemaphoreType.DMA],
    )(lg, g0, it0, ge, m1, m2,
      wov, why, wlg,
      bov.reshape(1, 1), bhy.reshape(1, 1), blg.reshape(1, 1))
    return out[1]


def _sc_gather(table, idx):
    b = idx.shape[0]
    d = table.shape[1]
    w = 128
    mesh = plsc.VectorSubcoreMesh(core_axis_name="c", subcore_axis_name="s")
    idx2 = idx.reshape(1, b)

    @pl.kernel(out_type=jax.ShapeDtypeStruct((b, d), jnp.float32),
               mesh=mesh)
    def k(tab_hbm, i_hbm, o_hbm):
        def body(i_vmem, o_vmem):
            pltpu.sync_copy(tab_hbm.at[i_vmem.at[0]], o_vmem)

        pltpu.emit_pipeline(
            body,
            grid=(b // w,),
            in_specs=[pl.BlockSpec((1, w), lambda i: (0, i))],
            out_specs=[pl.BlockSpec((w, d), lambda i: (i, 0))],
            core_axis_name=("c", "s"),
            dimension_semantics=(pltpu.PARALLEL,),
        )(i_hbm, o_hbm)

    return k(table, idx2)


def _dot_body(g_ref, i_ref, o_ref):
    o_ref[...] = jnp.sum(g_ref[...] * i_ref[...], axis=1, keepdims=True)


def _dot(gs, isel, bm):
    b, d = gs.shape
    out = pl.pallas_call(
        _dot_body,
        grid=(b // bm,),
        in_specs=[pl.BlockSpec((bm, d), lambda i: (i, 0)),
                  pl.BlockSpec((bm, d), lambda i: (i, 0))],
        out_specs=pl.BlockSpec((bm, 1), lambda i: (i, 0)),
        out_shape=jax.ShapeDtypeStruct((b, 1), jnp.float32),
    )(gs, isel)
    return out.reshape(b)


def kernel(user_table, item_table, group_table, user_hyper, item_hyper,
           full_hyper, overlap_graph, lgcn_graph, W_agg, b_agg,
           W_ov, b_ov, W_hy, b_hy, W_lg, b_lg,
           group_inputs, item_inputs):
    ge, m1, m2, _norm1, i_emb = _mk1(
        user_table, item_table, group_table, user_hyper, item_hyper,
        full_hyper, overlap_graph, W_agg, b_agg)

    i_sel = _sc_gather(i_emb, item_inputs)

    group_ui = _mk2(lgcn_graph, group_table, item_table, ge, m1, m2,
                    W_ov, W_hy, W_lg, b_ov, b_hy, b_lg)

    g_sel = _sc_gather(group_ui, group_inputs)
    return _dot(g_sel, i_sel, bm=2048)


# speedup vs baseline: 1.1609x; 1.0174x over previous
import jax
import jax.numpy as jnp
from jax.experimental import pallas as pl
from jax.experimental.pallas import tpu as pltpu
from jax.experimental.pallas import tpu_sc as plsc


def _start(src, dst, sem):
    pltpu.make_async_copy(src, dst, sem).start()


def _wait(src, dst, sem):
    pltpu.make_async_copy(src, dst, sem).wait()


def _mk1(u0, it0, g0, uh, ih, fht, a, wagg, bagg):
    nu, d = u0.shape
    ni = it0.shape[0]
    g = g0.shape[0]
    bm_g = 40
    bk = 80

    def body(u0_hbm, it0_hbm, g0_hbm, uh_hbm, ih_hbm, fht_hbm, a_hbm,
             wagg_hbm, bagg_hbm,
             oge_hbm, omsg1_hbm, omsg2_hbm, oiemb_hbm,
             s_a, s_u0, s_it0, s_g, s_ge, s_norm, s_it2, s_iemb,
             s_w, s_b, sem):
        first = ((u0_hbm, s_u0), (it0_hbm, s_it0), (g0_hbm, s_g),
                 (a_hbm, s_a), (wagg_hbm.at[0], s_w), (bagg_hbm.at[0:1], s_b))
        for src, dst in first:
            _start(src, dst, sem)
        for src, dst in first:
            _wait(src, dst, sem)

        gv = s_g[...]
        c1 = jnp.dot(s_a[...], gv, preferred_element_type=jnp.float32)
        c2 = jnp.dot(s_a[...], c1, preferred_element_type=jnp.float32)
        s_ge[...] = gv + c1 + c2
        _start(s_ge, oge_hbm, sem)
        _wait(s_ge, oge_hbm, sem)

        def make_msg_body(u_ref, it_ref):
            def msg_body(uh_ref, ih_ref, ge_ref, o_ref):
                um = jnp.dot(uh_ref[...], u_ref[...],
                             preferred_element_type=jnp.float32)
                im = jnp.dot(ih_ref[...], it_ref[...],
                             preferred_element_type=jnp.float32)
                ig = im * ge_ref[...]
                w = s_w[...]
                o_ref[...] = (jnp.dot(um, w[0:d],
                                      preferred_element_type=jnp.float32)
                              + jnp.dot(im, w[d:2 * d],
                                        preferred_element_type=jnp.float32)
                              + jnp.dot(ig, w[2 * d:3 * d],
                                        preferred_element_type=jnp.float32)
                              + s_b[...])
            return msg_body

        def run_msg(u_ref, it_ref, omsg_hbm):
            pltpu.emit_pipeline(
                make_msg_body(u_ref, it_ref),
                grid=(g // bm_g,),
                in_specs=[pl.BlockSpec((bm_g, nu), lambda i: (i, 0)),
                          pl.BlockSpec((bm_g, ni), lambda i: (i, 0)),
                          pl.BlockSpec((bm_g, d), lambda i: (i, 0))],
                out_specs=[pl.BlockSpec((bm_g, d), lambda i: (i, 0))],
            )(uh_hbm, ih_hbm, oge_hbm, omsg_hbm)

        run_msg(s_u0, s_it0, omsg1_hbm)

        s_norm[...] = jnp.zeros((nu + ni, d), jnp.float32)

        def norm_body(fht_ref, msg_ref):
            s_norm[...] += jax.lax.dot_general(
                fht_ref[...], msg_ref[...], (((0,), (0,)), ((), ())),
                preferred_element_type=jnp.float32)

        pltpu.emit_pipeline(
            norm_body,
            grid=(g // bk,),
            in_specs=[pl.BlockSpec((bk, nu + ni), lambda i: (i, 0)),
                      pl.BlockSpec((bk, d), lambda i: (i, 0))],
            out_specs=[],
        )(fht_hbm, omsg1_hbm)

        second = ((wagg_hbm.at[1], s_w), (bagg_hbm.at[1:2], s_b))
        for src, dst in second:
            _start(src, dst, sem)
        for src, dst in second:
            _wait(src, dst, sem)

        run_msg(s_norm.at[0:nu], s_norm.at[nu:nu + ni], omsg2_hbm)

        s_it2[...] = s_it0[...] + s_norm[nu:nu + ni, :]

        s_norm[...] = jnp.zeros((nu + ni, d), jnp.float32)
        pltpu.emit_pipeline(
            norm_body,
            grid=(g // bk,),
            in_specs=[pl.BlockSpec((bk, nu + ni), lambda i: (i, 0)),
                      pl.BlockSpec((bk, d), lambda i: (i, 0))],
            out_specs=[],
        )(fht_hbm, omsg2_hbm)

        v = s_it2[...] + s_norm[nu:nu + ni, :]
        s_iemb[...] = jnp.concatenate([v, jnp.zeros_like(v)], axis=1)
        _start(s_iemb, oiemb_hbm, sem)
        _wait(s_iemb, oiemb_hbm, sem)

    anyspec = pl.BlockSpec(memory_space=pltpu.MemorySpace.HBM)
    f32 = jnp.float32
    out = pl.pallas_call(
        body,
        in_specs=[anyspec] * 9,
        out_specs=[anyspec] * 4,
        out_shape=(jax.ShapeDtypeStruct((g, d), f32),
                   jax.ShapeDtypeStruct((g, d), f32),
                   jax.ShapeDtypeStruct((g, d), f32),
                   jax.ShapeDtypeStruct((ni, 2 * d), f32)),
        scratch_shapes=[pltpu.VMEM((g, g), f32),
                        pltpu.VMEM((nu, d), f32),
                        pltpu.VMEM((ni, d), f32),
                        pltpu.VMEM((g, d), f32),
                        pltpu.VMEM((g, d), f32),
                        pltpu.VMEM((nu + ni, d), f32),
                        pltpu.VMEM((ni, d), f32),
                        pltpu.VMEM((ni, 2 * d), f32),
                        pltpu.VMEM((3 * d, d), f32),
                        pltpu.VMEM((1, d), f32),
                        pltpu.SemaphoreType.DMA],
    )(u0, it0, g0, uh, ih, fht, a, wagg, bagg)
    return out


def _mk2(lg, g0, it0, ge, m1, m2, wov, why, wlg, bov, bhy, blg):
    g, d = g0.shape
    nlg = lg.shape[0]
    bm = 200

    def body(lg_hbm, g0_hbm, it0_hbm, ge_hbm, m1_hbm, m2_hbm,
             wov_hbm, why_hbm, wlg_hbm, bov_hbm, bhy_hbm, blg_hbm,
             oc1_hbm, ogui_hbm,
             s_e0, s_c1, s_wov, s_why, s_wlg, s_bov, s_bhy, s_blg, sem):
        first = ((g0_hbm, s_e0.at[0:g]), (it0_hbm.at[0:nlg - g], s_e0.at[g:nlg]),
                 (wov_hbm, s_wov), (why_hbm, s_why), (wlg_hbm, s_wlg),
                 (bov_hbm, s_bov), (bhy_hbm, s_bhy), (blg_hbm, s_blg))
        for src, dst in first:
            _start(src, dst, sem)
        for src, dst in first:
            _wait(src, dst, sem)

        def lg1_body(lg_ref, o_ref):
            o_ref[...] = jnp.dot(lg_ref[...], s_e0[...],
                                 preferred_element_type=jnp.float32)

        pltpu.emit_pipeline(
            lg1_body,
            grid=(nlg // bm,),
            in_specs=[pl.BlockSpec((bm, nlg), lambda i: (i, 0))],
            out_specs=[pl.BlockSpec((bm, d), lambda i: (i, 0))],
        )(lg_hbm, oc1_hbm)

        _start(oc1_hbm, s_c1, sem)
        _wait(oc1_hbm, s_c1, sem)

        def fuse_body(lg_ref, ge_ref, m1_ref, m2_ref, g0_ref, c1_ref, o_ref):
            c2 = jnp.dot(lg_ref[...], s_c1[...],
                         preferred_element_type=jnp.float32)
            ge_v = ge_ref[...]
            he = ge_v + m1_ref[...] + m2_ref[...]
            lgx = (g0_ref[...] + c1_ref[...] + c2) * (1.0 / 3.0)
            co = jax.nn.sigmoid(
                jnp.dot(ge_v, s_wov[...], preferred_element_type=jnp.float32)
                + s_bov[...])
            ch = jax.nn.sigmoid(
                jnp.dot(he, s_why[...], preferred_element_type=jnp.float32)
                + s_bhy[...])
            cl = jax.nn.sigmoid(
                jnp.dot(lgx, s_wlg[...], preferred_element_type=jnp.float32)
                + s_blg[...])
            v = co * ge_v + ch * he + cl * lgx
            o_ref[...] = jnp.concatenate([v, jnp.zeros_like(v)], axis=1)

        pltpu.emit_pipeline(
            fuse_body,
            grid=(g // bm,),
            in_specs=[pl.BlockSpec((bm, nlg), lambda i: (i, 0)),
                      pl.BlockSpec((bm, d), lambda i: (i, 0)),
                      pl.BlockSpec((bm, d), lambda i: (i, 0)),
                      pl.BlockSpec((bm, d), lambda i: (i, 0)),
                      pl.BlockSpec((bm, d), lambda i: (i, 0)),
                      pl.BlockSpec((bm, d), lambda i: (i, 0))],
            out_specs=[pl.BlockSpec((bm, 2 * d), lambda i: (i, 0))],
        )(lg_hbm, ge_hbm, m1_hbm, m2_hbm, g0_hbm, oc1_hbm, ogui_hbm)

    anyspec = pl.BlockSpec(memory_space=pltpu.MemorySpace.HBM)
    f32 = jnp.float32
    out = pl.pallas_call(
        body,
        in_specs=[anyspec] * 12,
        out_specs=[anyspec] * 2,
        out_shape=(jax.ShapeDtypeStruct((nlg, d), f32),
                   jax.ShapeDtypeStruct((g, 2 * d), f32)),
        scratch_shapes=[pltpu.VMEM((nlg, d), f32),
                        pltpu.VMEM((nlg, d), f32),
                        pltpu.VMEM((d, 1), f32),
                        pltpu.VMEM((d, 1), f32),
                        pltpu.VMEM((d, 1), f32),
                        pltpu.VMEM((1, 1), f32),
                        pltpu.VMEM((1, 1), f32),
                        pltpu.VMEM((1, 1), f32),
                        pltpu.SemaphoreType.DMA],
    )(lg, g0, it0, ge, m1, m2,
      wov, why, wlg,
      bov.reshape(1, 1), bhy.reshape(1, 1), blg.reshape(1, 1))
    return out[1]


def _sc_gather(table, idx):
    b = idx.shape[0]
    d = table.shape[1]
    w = 128
    mesh = plsc.VectorSubcoreMesh(core_axis_name="c", subcore_axis_name="s")
    idx2 = idx.reshape(1, b)

    @pl.kernel(out_type=jax.ShapeDtypeStruct((b, d), jnp.float32),
               mesh=mesh)
    def k(tab_hbm, i_hbm, o_hbm):
        def body(i_vmem, o_vmem):
            pltpu.sync_copy(tab_hbm.at[i_vmem.at[0]], o_vmem)

        pltpu.emit_pipeline(
            body,
            grid=(b // w,),
            in_specs=[pl.BlockSpec((1, w), lambda i: (0, i))],
            out_specs=[pl.BlockSpec((w, d), lambda i: (i, 0))],
            core_axis_name=("c", "s"),
            dimension_semantics=(pltpu.PARALLEL,),
        )(i_hbm, o_hbm)

    return k(table, idx2)


def _dot_body(g_ref, i_ref, o_ref):
    o_ref[...] = jnp.sum(g_ref[...] * i_ref[...], axis=1, keepdims=True)


def _dot(gs, isel, bm):
    b, d = gs.shape
    out = pl.pallas_call(
        _dot_body,
        grid=(b // bm,),
        in_specs=[pl.BlockSpec((bm, d), lambda i: (i, 0)),
                  pl.BlockSpec((bm, d), lambda i: (i, 0))],
        out_specs=pl.BlockSpec((bm, 1), lambda i: (i, 0)),
        out_shape=jax.ShapeDtypeStruct((b, 1), jnp.float32),
    )(gs, isel)
    return out.reshape(b)


def kernel(user_table, item_table, group_table, user_hyper, item_hyper,
           full_hyper, overlap_graph, lgcn_graph, W_agg, b_agg,
           W_ov, b_ov, W_hy, b_hy, W_lg, b_lg,
           group_inputs, item_inputs):
    ge, m1, m2, i_emb = _mk1(
        user_table, item_table, group_table, user_hyper, item_hyper,
        full_hyper.T, overlap_graph, W_agg, b_agg)

    i_sel = _sc_gather(i_emb, item_inputs)

    group_ui = _mk2(lgcn_graph, group_table, item_table, ge, m1, m2,
                    W_ov, W_hy, W_lg, b_ov, b_hy, b_lg)

    g_sel = _sc_gather(group_ui, group_inputs)
    return _dot(g_sel, i_sel, bm=2048)


# speedup vs baseline: 1.2813x; 1.1037x over previous
import jax
import jax.numpy as jnp
from jax.experimental import pallas as pl
from jax.experimental.pallas import tpu as pltpu
from jax.experimental.pallas import tpu_sc as plsc


def _start(src, dst, sem):
    pltpu.make_async_copy(src, dst, sem).start()


def _wait(src, dst, sem):
    pltpu.make_async_copy(src, dst, sem).wait()


def _mk1(u0, it0, g0, uh, ih, fht, a, wagg, bagg):
    nu, d = u0.shape
    ni = it0.shape[0]
    g = g0.shape[0]
    bm_g = 40
    bk = 80

    def body(u0_hbm, it0_hbm, g0_hbm, uh_hbm, ih_hbm, fht_hbm, a_hbm,
             wagg_hbm, bagg_hbm,
             oge_hbm, omsg1_hbm, omsg2_hbm, oiemb_hbm,
             s_a, s_u0, s_it0, s_g, s_ge, s_norm, s_normt, s_it2, s_iemb,
             s_w, s_b, sem):
        first = ((u0_hbm, s_u0), (it0_hbm, s_it0), (g0_hbm, s_g),
                 (a_hbm, s_a), (wagg_hbm.at[0], s_w), (bagg_hbm.at[0:1], s_b))
        for src, dst in first:
            _start(src, dst, sem)
        for src, dst in first:
            _wait(src, dst, sem)

        gv = s_g[...]
        c1 = jnp.dot(s_a[...], gv, preferred_element_type=jnp.float32)
        c2 = jnp.dot(s_a[...], c1, preferred_element_type=jnp.float32)
        s_ge[...] = gv + c1 + c2
        _start(s_ge, oge_hbm, sem)
        _wait(s_ge, oge_hbm, sem)

        def make_msg_body(u_ref, it_ref):
            def msg_body(uh_ref, ih_ref, ge_ref, o_ref):
                um = jnp.dot(uh_ref[...], u_ref[...],
                             preferred_element_type=jnp.float32)
                im = jnp.dot(ih_ref[...], it_ref[...],
                             preferred_element_type=jnp.float32)
                ig = im * ge_ref[...]
                w = s_w[...]
                o_ref[...] = (jnp.dot(um, w[0:d],
                                      preferred_element_type=jnp.float32)
                              + jnp.dot(im, w[d:2 * d],
                                        preferred_element_type=jnp.float32)
                              + jnp.dot(ig, w[2 * d:3 * d],
                                        preferred_element_type=jnp.float32)
                              + s_b[...])
            return msg_body

        def run_msg(u_ref, it_ref, omsg_hbm):
            pltpu.emit_pipeline(
                make_msg_body(u_ref, it_ref),
                grid=(g // bm_g,),
                in_specs=[pl.BlockSpec((bm_g, nu), lambda i: (i, 0)),
                          pl.BlockSpec((bm_g, ni), lambda i: (i, 0)),
                          pl.BlockSpec((bm_g, d), lambda i: (i, 0))],
                out_specs=[pl.BlockSpec((bm_g, d), lambda i: (i, 0))],
            )(uh_hbm, ih_hbm, oge_hbm, omsg_hbm)

        run_msg(s_u0, s_it0, omsg1_hbm)

        s_normt[...] = jnp.zeros((d, nu + ni), jnp.float32)

        def norm_body(fht_ref, msg_ref):
            s_normt[...] += jax.lax.dot_general(
                msg_ref[...], fht_ref[...], (((0,), (0,)), ((), ())),
                preferred_element_type=jnp.float32)

        pltpu.emit_pipeline(
            norm_body,
            grid=(g // bk,),
            in_specs=[pl.BlockSpec((bk, nu + ni), lambda i: (i, 0)),
                      pl.BlockSpec((bk, d), lambda i: (i, 0))],
            out_specs=[],
        )(fht_hbm, omsg1_hbm)
        s_norm[...] = s_normt[...].T

        second = ((wagg_hbm.at[1], s_w), (bagg_hbm.at[1:2], s_b))
        for src, dst in second:
            _start(src, dst, sem)
        for src, dst in second:
            _wait(src, dst, sem)

        run_msg(s_norm.at[0:nu], s_norm.at[nu:nu + ni], omsg2_hbm)

        s_it2[...] = s_it0[...] + s_norm[nu:nu + ni, :]

        s_normt[...] = jnp.zeros((d, nu + ni), jnp.float32)
        pltpu.emit_pipeline(
            norm_body,
            grid=(g // bk,),
            in_specs=[pl.BlockSpec((bk, nu + ni), lambda i: (i, 0)),
                      pl.BlockSpec((bk, d), lambda i: (i, 0))],
            out_specs=[],
        )(fht_hbm, omsg2_hbm)

        v = s_it2[...] + s_normt[:, nu:nu + ni].T
        s_iemb[...] = jnp.concatenate([v, jnp.zeros_like(v)], axis=1)
        _start(s_iemb, oiemb_hbm, sem)
        _wait(s_iemb, oiemb_hbm, sem)

    anyspec = pl.BlockSpec(memory_space=pltpu.MemorySpace.HBM)
    f32 = jnp.float32
    out = pl.pallas_call(
        body,
        in_specs=[anyspec] * 9,
        out_specs=[anyspec] * 4,
        out_shape=(jax.ShapeDtypeStruct((g, d), f32),
                   jax.ShapeDtypeStruct((g, d), f32),
                   jax.ShapeDtypeStruct((g, d), f32),
                   jax.ShapeDtypeStruct((ni, 2 * d), f32)),
        scratch_shapes=[pltpu.VMEM((g, g), f32),
                        pltpu.VMEM((nu, d), f32),
                        pltpu.VMEM((ni, d), f32),
                        pltpu.VMEM((g, d), f32),
                        pltpu.VMEM((g, d), f32),
                        pltpu.VMEM((nu + ni, d), f32),
                        pltpu.VMEM((d, nu + ni), f32),
                        pltpu.VMEM((ni, d), f32),
                        pltpu.VMEM((ni, 2 * d), f32),
                        pltpu.VMEM((3 * d, d), f32),
                        pltpu.VMEM((1, d), f32),
                        pltpu.SemaphoreType.DMA],
    )(u0, it0, g0, uh, ih, fht, a, wagg, bagg)
    return out


def _mk2(lg, g0, it0, ge, m1, m2, wov, why, wlg, bov, bhy, blg):
    g, d = g0.shape
    nlg = lg.shape[0]
    bm = 200

    def body(lg_hbm, g0_hbm, it0_hbm, ge_hbm, m1_hbm, m2_hbm,
             wov_hbm, why_hbm, wlg_hbm, bov_hbm, bhy_hbm, blg_hbm,
             oc1_hbm, ogui_hbm,
             s_e0, s_c1, s_wov, s_why, s_wlg, s_bov, s_bhy, s_blg, sem):
        first = ((g0_hbm, s_e0.at[0:g]), (it0_hbm.at[0:nlg - g], s_e0.at[g:nlg]),
                 (wov_hbm, s_wov), (why_hbm, s_why), (wlg_hbm, s_wlg),
                 (bov_hbm, s_bov), (bhy_hbm, s_bhy), (blg_hbm, s_blg))
        for src, dst in first:
            _start(src, dst, sem)
        for src, dst in first:
            _wait(src, dst, sem)

        def lg1_body(lg_ref, o_ref):
            o_ref[...] = jnp.dot(lg_ref[...], s_e0[...],
                                 preferred_element_type=jnp.float32)

        pltpu.emit_pipeline(
            lg1_body,
            grid=(nlg // bm,),
            in_specs=[pl.BlockSpec((bm, nlg), lambda i: (i, 0))],
            out_specs=[pl.BlockSpec((bm, d), lambda i: (i, 0))],
        )(lg_hbm, oc1_hbm)

        _start(oc1_hbm, s_c1, sem)
        _wait(oc1_hbm, s_c1, sem)

        def fuse_body(lg_ref, ge_ref, m1_ref, m2_ref, g0_ref, c1_ref, o_ref):
            c2 = jnp.dot(lg_ref[...], s_c1[...],
                         preferred_element_type=jnp.float32)
            ge_v = ge_ref[...]
            he = ge_v + m1_ref[...] + m2_ref[...]
            lgx = (g0_ref[...] + c1_ref[...] + c2) * (1.0 / 3.0)
            co = jax.nn.sigmoid(
                jnp.dot(ge_v, s_wov[...], preferred_element_type=jnp.float32)
                + s_bov[...])
            ch = jax.nn.sigmoid(
                jnp.dot(he, s_why[...], preferred_element_type=jnp.float32)
                + s_bhy[...])
            cl = jax.nn.sigmoid(
                jnp.dot(lgx, s_wlg[...], preferred_element_type=jnp.float32)
                + s_blg[...])
            v = co * ge_v + ch * he + cl * lgx
            o_ref[...] = jnp.concatenate([v, jnp.zeros_like(v)], axis=1)

        pltpu.emit_pipeline(
            fuse_body,
            grid=(g // bm,),
            in_specs=[pl.BlockSpec((bm, nlg), lambda i: (i, 0)),
                      pl.BlockSpec((bm, d), lambda i: (i, 0)),
                      pl.BlockSpec((bm, d), lambda i: (i, 0)),
                      pl.BlockSpec((bm, d), lambda i: (i, 0)),
                      pl.BlockSpec((bm, d), lambda i: (i, 0)),
                      pl.BlockSpec((bm, d), lambda i: (i, 0))],
            out_specs=[pl.BlockSpec((bm, 2 * d), lambda i: (i, 0))],
        )(lg_hbm, ge_hbm, m1_hbm, m2_hbm, g0_hbm, oc1_hbm, ogui_hbm)

    anyspec = pl.BlockSpec(memory_space=pltpu.MemorySpace.HBM)
    f32 = jnp.float32
    out = pl.pallas_call(
        body,
        in_specs=[anyspec] * 12,
        out_specs=[anyspec] * 2,
        out_shape=(jax.ShapeDtypeStruct((nlg, d), f32),
                   jax.ShapeDtypeStruct((g, 2 * d), f32)),
        scratch_shapes=[pltpu.VMEM((nlg, d), f32),
                        pltpu.VMEM((nlg, d), f32),
                        pltpu.VMEM((d, 1), f32),
                        pltpu.VMEM((d, 1), f32),
                        pltpu.VMEM((d, 1), f32),
                        pltpu.VMEM((1, 1), f32),
                        pltpu.VMEM((1, 1), f32),
                        pltpu.VMEM((1, 1), f32),
                        pltpu.SemaphoreType.DMA],
    )(lg, g0, it0, ge, m1, m2,
      wov, why, wlg,
      bov.reshape(1, 1), bhy.reshape(1, 1), blg.reshape(1, 1))
    return out[1]


def _sc_gather(table, idx):
    b = idx.shape[0]
    d = table.shape[1]
    w = 128
    mesh = plsc.VectorSubcoreMesh(core_axis_name="c", subcore_axis_name="s")
    idx2 = idx.reshape(1, b)

    @pl.kernel(out_type=jax.ShapeDtypeStruct((b, d), jnp.float32),
               mesh=mesh)
    def k(tab_hbm, i_hbm, o_hbm):
        def body(i_vmem, o_vmem):
            pltpu.sync_copy(tab_hbm.at[i_vmem.at[0]], o_vmem)

        pltpu.emit_pipeline(
            body,
            grid=(b // w,),
            in_specs=[pl.BlockSpec((1, w), lambda i: (0, i))],
            out_specs=[pl.BlockSpec((w, d), lambda i: (i, 0))],
            core_axis_name=("c", "s"),
            dimension_semantics=(pltpu.PARALLEL,),
        )(i_hbm, o_hbm)

    return k(table, idx2)


def _dot_body(g_ref, i_ref, o_ref):
    o_ref[...] = jnp.sum(g_ref[...] * i_ref[...], axis=1, keepdims=True)


def _dot(gs, isel, bm):
    b, d = gs.shape
    out = pl.pallas_call(
        _dot_body,
        grid=(b // bm,),
        in_specs=[pl.BlockSpec((bm, d), lambda i: (i, 0)),
                  pl.BlockSpec((bm, d), lambda i: (i, 0))],
        out_specs=pl.BlockSpec((bm, 1), lambda i: (i, 0)),
        out_shape=jax.ShapeDtypeStruct((b, 1), jnp.float32),
    )(gs, isel)
    return out.reshape(b)


def kernel(user_table, item_table, group_table, user_hyper, item_hyper,
           full_hyper, overlap_graph, lgcn_graph, W_agg, b_agg,
           W_ov, b_ov, W_hy, b_hy, W_lg, b_lg,
           group_inputs, item_inputs):
    ge, m1, m2, i_emb = _mk1(
        user_table, item_table, group_table, user_hyper, item_hyper,
        full_hyper.T, overlap_graph, W_agg, b_agg)

    i_sel = _sc_gather(i_emb, item_inputs)

    group_ui = _mk2(lgcn_graph, group_table, item_table, ge, m1, m2,
                    W_ov, W_hy, W_lg, b_ov, b_hy, b_lg)

    g_sel = _sc_gather(group_ui, group_inputs)
    return _dot(g_sel, i_sel, bm=2048)


# speedup vs baseline: 1.5646x; 1.2211x over previous
import jax
import jax.numpy as jnp
from jax.experimental import pallas as pl
from jax.experimental.pallas import tpu as pltpu
from jax.experimental.pallas import tpu_sc as plsc


def _start(src, dst, sem):
    pltpu.make_async_copy(src, dst, sem).start()


def _wait(src, dst, sem):
    pltpu.make_async_copy(src, dst, sem).wait()


def _mk1(u0t, it0t, g0t, uh, ih, fht, a, waggt, bagg, nlg):
    d, nu = u0t.shape
    ni = it0t.shape[1]
    g = g0t.shape[1]
    bk = 40

    def body(u0t_hbm, it0t_hbm, g0t_hbm, uh_hbm, ih_hbm, fht_hbm, a_hbm,
             waggt_hbm, bagg_hbm,
             oge_hbm, omsg1_hbm, omsg2_hbm, oiemb_hbm, oe0_hbm,
             s_a, s_u0t, s_it0t, s_gt, s_g, s_ge, s_norm, s_normt,
             s_it2, s_iemb, s_wt, s_w, s_b, sem):
        first = ((u0t_hbm, s_u0t), (it0t_hbm, s_it0t),
                 (g0t_hbm, s_gt), (a_hbm, s_a),
                 (waggt_hbm.at[0], s_wt), (bagg_hbm.at[0:1], s_b))
        for src, dst in first:
            _start(src, dst, sem)
        for src, dst in first:
            _wait(src, dst, sem)

        s_norm[0:nu, :] = s_u0t[...].T
        s_norm[nu:nu + ni, :] = s_it0t[...].T
        s_g[...] = s_gt[...].T
        s_w[...] = s_wt[...].T

        gv = s_g[...]
        c1 = jnp.dot(s_a[...], gv, preferred_element_type=jnp.float32)
        c2 = jnp.dot(s_a[...], c1, preferred_element_type=jnp.float32)
        s_ge[...] = gv + c1 + c2
        _start(s_ge, oge_hbm, sem)

        _start(s_g, oe0_hbm.at[0:g], sem)
        _start(s_norm.at[nu:nu + (nlg - g)], oe0_hbm.at[g:nlg], sem)
        _wait(s_ge, oge_hbm, sem)
        _wait(s_g, oe0_hbm.at[0:g], sem)
        _wait(s_norm.at[nu:nu + (nlg - g)], oe0_hbm.at[g:nlg], sem)

        s_it2[...] = s_norm[nu:nu + ni, :]

        def make_layer_body(u_ref, it_ref):
            def layer_body(uh_ref, ih_ref, ge_ref, fht_ref, omsg_ref):
                um = jnp.dot(uh_ref[...], u_ref[...],
                             preferred_element_type=jnp.float32)
                im = jnp.dot(ih_ref[...], it_ref[...],
                             preferred_element_type=jnp.float32)
                ig = im * ge_ref[...]
                w = s_w[...]
                msgb = (jnp.dot(um, w[0:d],
                                preferred_element_type=jnp.float32)
                        + jnp.dot(im, w[d:2 * d],
                                  preferred_element_type=jnp.float32)
                        + jnp.dot(ig, w[2 * d:3 * d],
                                  preferred_element_type=jnp.float32)
                        + s_b[...])
                omsg_ref[...] = msgb
                s_normt[...] += jax.lax.dot_general(
                    msgb, fht_ref[...], (((0,), (0,)), ((), ())),
                    preferred_element_type=jnp.float32)
            return layer_body

        def run_layer(u_ref, it_ref, omsg_hbm):
            pltpu.emit_pipeline(
                make_layer_body(u_ref, it_ref),
                grid=(g // bk,),
                in_specs=[pl.BlockSpec((bk, nu), lambda i: (i, 0)),
                          pl.BlockSpec((bk, ni), lambda i: (i, 0)),
                          pl.BlockSpec((bk, d), lambda i: (i, 0)),
                          pl.BlockSpec((bk, nu + ni), lambda i: (i, 0))],
                out_specs=[pl.BlockSpec((bk, d), lambda i: (i, 0))],
            )(uh_hbm, ih_hbm, oge_hbm, fht_hbm, omsg_hbm)

        s_normt[...] = jnp.zeros((d, nu + ni), jnp.float32)
        run_layer(s_norm.at[0:nu], s_norm.at[nu:nu + ni], omsg1_hbm)
        s_norm[...] = s_normt[...].T
        s_it2[...] += s_norm[nu:nu + ni, :]

        second = ((waggt_hbm.at[1], s_wt), (bagg_hbm.at[1:2], s_b))
        for src, dst in second:
            _start(src, dst, sem)
        for src, dst in second:
            _wait(src, dst, sem)
        s_w[...] = s_wt[...].T

        s_normt[...] = jnp.zeros((d, nu + ni), jnp.float32)
        run_layer(s_norm.at[0:nu], s_norm.at[nu:nu + ni], omsg2_hbm)

        v = s_it2[...] + s_normt[:, nu:nu + ni].T
        s_iemb[...] = jnp.concatenate([v, jnp.zeros_like(v)], axis=1)
        _start(s_iemb, oiemb_hbm, sem)
        _wait(s_iemb, oiemb_hbm, sem)

    anyspec = pl.BlockSpec(memory_space=pltpu.MemorySpace.HBM)
    f32 = jnp.float32
    out = pl.pallas_call(
        body,
        in_specs=[anyspec] * 9,
        out_specs=[anyspec] * 5,
        out_shape=(jax.ShapeDtypeStruct((g, d), f32),
                   jax.ShapeDtypeStruct((g, d), f32),
                   jax.ShapeDtypeStruct((g, d), f32),
                   jax.ShapeDtypeStruct((ni, 2 * d), f32),
                   jax.ShapeDtypeStruct((nlg, d), f32)),
        scratch_shapes=[pltpu.VMEM((g, g), f32),
                        pltpu.VMEM((d, nu), f32),
                        pltpu.VMEM((d, ni), f32),
                        pltpu.VMEM((d, g), f32),
                        pltpu.VMEM((g, d), f32),
                        pltpu.VMEM((g, d), f32),
                        pltpu.VMEM((nu + ni, d), f32),
                        pltpu.VMEM((d, nu + ni), f32),
                        pltpu.VMEM((ni, d), f32),
                        pltpu.VMEM((ni, 2 * d), f32),
                        pltpu.VMEM((d, 3 * d), f32),
                        pltpu.VMEM((3 * d, d), f32),
                        pltpu.VMEM((1, d), f32),
                        pltpu.SemaphoreType.DMA],
    )(u0t, it0t, g0t, uh, ih, fht, a, waggt, bagg)
    return out


def _mk2(lg, e0, ge, m1, m2, wovt, whyt, wlgt, bov, bhy, blg):
    g, d = ge.shape
    nlg = lg.shape[0]
    bm = 400

    def body(lg_hbm, e0_hbm, ge_hbm, m1_hbm, m2_hbm,
             wovt_hbm, whyt_hbm, wlgt_hbm, bov_hbm, bhy_hbm, blg_hbm,
             oc1_hbm, ogui_hbm,
             s_e0, s_c1, s_wovt, s_whyt, s_wlgt, s_wov, s_why, s_wlg,
             s_bov, s_bhy, s_blg, sem):
        first = ((e0_hbm, s_e0),
                 (wovt_hbm, s_wovt), (whyt_hbm, s_whyt), (wlgt_hbm, s_wlgt),
                 (bov_hbm, s_bov), (bhy_hbm, s_bhy), (blg_hbm, s_blg))
        for src, dst in first:
            _start(src, dst, sem)
        for src, dst in first:
            _wait(src, dst, sem)
        s_wov[...] = s_wovt[...].T
        s_why[...] = s_whyt[...].T
        s_wlg[...] = s_wlgt[...].T

        def lg1_body(lg_ref, o_ref):
            o_ref[...] = jnp.dot(lg_ref[...], s_e0[...],
                                 preferred_element_type=jnp.float32)

        pltpu.emit_pipeline(
            lg1_body,
            grid=(nlg // bm,),
            in_specs=[pl.BlockSpec((bm, nlg), lambda i: (i, 0))],
            out_specs=[pl.BlockSpec((bm, d), lambda i: (i, 0))],
        )(lg_hbm, oc1_hbm)

        _start(oc1_hbm, s_c1, sem)
        _wait(oc1_hbm, s_c1, sem)

        def fuse_body(lg_ref, ge_ref, m1_ref, m2_ref, g0_ref, c1_ref, o_ref):
            c2 = jnp.dot(lg_ref[...], s_c1[...],
                         preferred_element_type=jnp.float32)
            ge_v = ge_ref[...]
            he = ge_v + m1_ref[...] + m2_ref[...]
            lgx = (g0_ref[...] + c1_ref[...] + c2) * (1.0 / 3.0)
            co = jax.nn.sigmoid(
                jnp.dot(ge_v, s_wov[...], preferred_element_type=jnp.float32)
                + s_bov[...])
            ch = jax.nn.sigmoid(
                jnp.dot(he, s_why[...], preferred_element_type=jnp.float32)
                + s_bhy[...])
            cl = jax.nn.sigmoid(
                jnp.dot(lgx, s_wlg[...], preferred_element_type=jnp.float32)
                + s_blg[...])
            v = co * ge_v + ch * he + cl * lgx
            o_ref[...] = jnp.concatenate([v, jnp.zeros_like(v)], axis=1)

        pltpu.emit_pipeline(
            fuse_body,
            grid=(g // bm,),
            in_specs=[pl.BlockSpec((bm, nlg), lambda i: (i, 0)),
                      pl.BlockSpec((bm, d), lambda i: (i, 0)),
                      pl.BlockSpec((bm, d), lambda i: (i, 0)),
                      pl.BlockSpec((bm, d), lambda i: (i, 0)),
                      pl.BlockSpec((bm, d), lambda i: (i, 0)),
                      pl.BlockSpec((bm, d), lambda i: (i, 0))],
            out_specs=[pl.BlockSpec((bm, 2 * d), lambda i: (i, 0))],
        )(lg_hbm, ge_hbm, m1_hbm, m2_hbm, e0_hbm, oc1_hbm, ogui_hbm)

    anyspec = pl.BlockSpec(memory_space=pltpu.MemorySpace.HBM)
    f32 = jnp.float32
    out = pl.pallas_call(
        body,
        in_specs=[anyspec] * 11,
        out_specs=[anyspec] * 2,
        out_shape=(jax.ShapeDtypeStruct((nlg, d), f32),
                   jax.ShapeDtypeStruct((g, 2 * d), f32)),
        scratch_shapes=[pltpu.VMEM((nlg, d), f32),
                        pltpu.VMEM((nlg, d), f32),
                        pltpu.VMEM((1, d), f32),
                        pltpu.VMEM((1, d), f32),
                        pltpu.VMEM((1, d), f32),
                        pltpu.VMEM((d, 1), f32),
                        pltpu.VMEM((d, 1), f32),
                        pltpu.VMEM((d, 1), f32),
                        pltpu.VMEM((1, 1), f32),
                        pltpu.VMEM((1, 1), f32),
                        pltpu.VMEM((1, 1), f32),
                        pltpu.SemaphoreType.DMA],
    )(lg, e0, ge, m1, m2,
      wovt, whyt, wlgt,
      bov.reshape(1, 1), bhy.reshape(1, 1), blg.reshape(1, 1))
    return out[1]


def _sc_gather_pair(gtab, itab, gidx, iidx):
    b = gidx.shape[0]
    d = gtab.shape[1]
    w = 128
    mesh = plsc.VectorSubcoreMesh(core_axis_name="c", subcore_axis_name="s")
    gi2 = gidx.reshape(1, b)
    ii2 = iidx.reshape(1, b)

    @pl.kernel(out_type=(jax.ShapeDtypeStruct((b, d), jnp.float32),
                         jax.ShapeDtypeStruct((b, d), jnp.float32)),
               mesh=mesh,
               scratch_types=[pltpu.SemaphoreType.DMA,
                              pltpu.SemaphoreType.DMA])
    def k(gtab_hbm, itab_hbm, gi_hbm, ii_hbm, og_hbm, oi_hbm, sem1, sem2):
        def body(gi_vmem, ii_vmem, og_vmem, oi_vmem):
            cg = pltpu.make_async_copy(gtab_hbm.at[gi_vmem.at[0]], og_vmem,
                                       sem1)
            ci = pltpu.make_async_copy(itab_hbm.at[ii_vmem.at[0]], oi_vmem,
                                       sem2)
            cg.start()
            ci.start()
            cg.wait()
            ci.wait()

        pltpu.emit_pipeline(
            body,
            grid=(b // w,),
            in_specs=[pl.BlockSpec((1, w), lambda i: (0, i)),
                      pl.BlockSpec((1, w), lambda i: (0, i))],
            out_specs=[pl.BlockSpec((w, d), lambda i: (i, 0)),
                       pl.BlockSpec((w, d), lambda i: (i, 0))],
            core_axis_name=("c", "s"),
            dimension_semantics=(pltpu.PARALLEL,),
        )(gi_hbm, ii_hbm, og_hbm, oi_hbm)

    return k(gtab, itab, gi2, ii2)


def _dot_body(g_ref, i_ref, o_ref):
    s = jnp.sum(g_ref[...] * i_ref[...], axis=1)
    o_ref[...] = s.reshape(1, s.shape[0])


def _dot(gs, isel, bm):
    b, d = gs.shape
    out = pl.pallas_call(
        _dot_body,
        grid=(b // bm,),
        in_specs=[pl.BlockSpec((bm, d), lambda i: (i, 0)),
                  pl.BlockSpec((bm, d), lambda i: (i, 0))],
        out_specs=pl.BlockSpec((1, bm), lambda i: (0, i)),
        out_shape=jax.ShapeDtypeStruct((1, b), jnp.float32),
    )(gs, isel)
    return out.reshape(b)


def kernel(user_table, item_table, group_table, user_hyper, item_hyper,
           full_hyper, overlap_graph, lgcn_graph, W_agg, b_agg,
           W_ov, b_ov, W_hy, b_hy, W_lg, b_lg,
           group_inputs, item_inputs):
    nlg = lgcn_graph.shape[0]

    ge, m1, m2, i_emb, e0 = _mk1(
        user_table.T, item_table.T, group_table.T, user_hyper, item_hyper,
        full_hyper.T, overlap_graph, W_agg.transpose(0, 2, 1), b_agg, nlg)

    group_ui = _mk2(lgcn_graph, e0, ge, m1, m2,
                    W_ov.T, W_hy.T, W_lg.T, b_ov, b_hy, b_lg)

    g_sel, i_sel = _sc_gather_pair(group_ui, i_emb, group_inputs, item_inputs)
    return _dot(g_sel, i_sel, bm=4096)


# speedup vs baseline: 1.7990x; 1.1498x over previous
import jax
import jax.numpy as jnp
from jax.experimental import pallas as pl
from jax.experimental.pallas import tpu as pltpu
from jax.experimental.pallas import tpu_sc as plsc


def _start(src, dst, sem):
    pltpu.make_async_copy(src, dst, sem).start()


def _wait(src, dst, sem):
    pltpu.make_async_copy(src, dst, sem).wait()


def _mk1(u0t, it0t, g0t, uh, ih, fht, a, waggt, bagg, nlg):
    d, nu = u0t.shape
    ni = it0t.shape[1]
    g = g0t.shape[1]
    bk = 80

    def body(u0t_hbm, it0t_hbm, g0t_hbm, uh_hbm, ih_hbm, fht_hbm, a_hbm,
             waggt_hbm, bagg_hbm,
             oge_hbm, omsg1_hbm, omsg2_hbm, oiemb_hbm, oe0_hbm,
             s_a, s_u0t, s_it0t, s_gt, s_g, s_ge, s_norm, s_normt,
             s_it2, s_iemb, s_wt, s_w, s_b, sem):
        first = ((u0t_hbm, s_u0t), (it0t_hbm, s_it0t),
                 (g0t_hbm, s_gt), (a_hbm, s_a),
                 (waggt_hbm.at[0], s_wt), (bagg_hbm.at[0:1], s_b))
        for src, dst in first:
            _start(src, dst, sem)
        for src, dst in first:
            _wait(src, dst, sem)

        s_norm[0:nu, :] = s_u0t[...].T
        s_norm[nu:nu + ni, :] = s_it0t[...].T
        s_g[...] = s_gt[...].T
        s_w[...] = s_wt[...].T

        gv = s_g[...]
        c1 = jnp.dot(s_a[...], gv, preferred_element_type=jnp.float32)
        c2 = jnp.dot(s_a[...], c1, preferred_element_type=jnp.float32)
        s_ge[...] = gv + c1 + c2
        _start(s_ge, oge_hbm, sem)

        _start(s_g, oe0_hbm.at[0:g], sem)
        _start(s_norm.at[nu:nu + (nlg - g)], oe0_hbm.at[g:nlg], sem)
        _wait(s_ge, oge_hbm, sem)
        _wait(s_g, oe0_hbm.at[0:g], sem)
        _wait(s_norm.at[nu:nu + (nlg - g)], oe0_hbm.at[g:nlg], sem)

        s_it2[...] = s_norm[nu:nu + ni, :]

        def make_layer_body(u_ref, it_ref):
            def layer_body(uh_ref, ih_ref, ge_ref, fht_ref, omsg_ref):
                um = jnp.dot(uh_ref[...], u_ref[...],
                             preferred_element_type=jnp.float32)
                im = jnp.dot(ih_ref[...], it_ref[...],
                             preferred_element_type=jnp.float32)
                ig = im * ge_ref[...]
                w = s_w[...]
                msgb = (jnp.dot(um, w[0:d],
                                preferred_element_type=jnp.float32)
                        + jnp.dot(im, w[d:2 * d],
                                  preferred_element_type=jnp.float32)
                        + jnp.dot(ig, w[2 * d:3 * d],
                                  preferred_element_type=jnp.float32)
                        + s_b[...])
                omsg_ref[...] = msgb
                s_normt[...] += jax.lax.dot_general(
                    msgb, fht_ref[...], (((0,), (0,)), ((), ())),
                    preferred_element_type=jnp.float32)
            return layer_body

        def run_layer(u_ref, it_ref, omsg_hbm):
            pltpu.emit_pipeline(
                make_layer_body(u_ref, it_ref),
                grid=(g // bk,),
                in_specs=[pl.BlockSpec((bk, nu), lambda i: (i, 0)),
                          pl.BlockSpec((bk, ni), lambda i: (i, 0)),
                          pl.BlockSpec((bk, d), lambda i: (i, 0)),
                          pl.BlockSpec((bk, nu + ni), lambda i: (i, 0))],
                out_specs=[pl.BlockSpec((bk, d), lambda i: (i, 0))],
            )(uh_hbm, ih_hbm, oge_hbm, fht_hbm, omsg_hbm)

        s_normt[...] = jnp.zeros((d, nu + ni), jnp.float32)
        run_layer(s_norm.at[0:nu], s_norm.at[nu:nu + ni], omsg1_hbm)
        s_norm[...] = s_normt[...].T
        s_it2[...] += s_norm[nu:nu + ni, :]

        second = ((waggt_hbm.at[1], s_wt), (bagg_hbm.at[1:2], s_b))
        for src, dst in second:
            _start(src, dst, sem)
        for src, dst in second:
            _wait(src, dst, sem)
        s_w[...] = s_wt[...].T

        s_normt[...] = jnp.zeros((d, nu + ni), jnp.float32)
        run_layer(s_norm.at[0:nu], s_norm.at[nu:nu + ni], omsg2_hbm)

        v = s_it2[...] + s_normt[:, nu:nu + ni].T
        s_iemb[...] = jnp.concatenate([v, jnp.zeros_like(v)], axis=1)
        _start(s_iemb, oiemb_hbm, sem)
        _wait(s_iemb, oiemb_hbm, sem)

    anyspec = pl.BlockSpec(memory_space=pltpu.MemorySpace.HBM)
    f32 = jnp.float32
    out = pl.pallas_call(
        body,
        in_specs=[anyspec] * 9,
        out_specs=[anyspec] * 5,
        out_shape=(jax.ShapeDtypeStruct((g, d), f32),
                   jax.ShapeDtypeStruct((g, d), f32),
                   jax.ShapeDtypeStruct((g, d), f32),
                   jax.ShapeDtypeStruct((ni, 2 * d), f32),
                   jax.ShapeDtypeStruct((nlg, d), f32)),
        scratch_shapes=[pltpu.VMEM((g, g), f32),
                        pltpu.VMEM((d, nu), f32),
                        pltpu.VMEM((d, ni), f32),
                        pltpu.VMEM((d, g), f32),
                        pltpu.VMEM((g, d), f32),
                        pltpu.VMEM((g, d), f32),
                        pltpu.VMEM((nu + ni, d), f32),
                        pltpu.VMEM((d, nu + ni), f32),
                        pltpu.VMEM((ni, d), f32),
                        pltpu.VMEM((ni, 2 * d), f32),
                        pltpu.VMEM((d, 3 * d), f32),
                        pltpu.VMEM((3 * d, d), f32),
                        pltpu.VMEM((1, d), f32),
                        pltpu.SemaphoreType.DMA],
    )(u0t, it0t, g0t, uh, ih, fht, a, waggt, bagg)
    return out


def _mk2(lg, e0, ge, m1, m2, wovt, whyt, wlgt, bov, bhy, blg):
    g, d = ge.shape
    nlg = lg.shape[0]
    bm = 400

    def body(lg_hbm, e0_hbm, ge_hbm, m1_hbm, m2_hbm,
             wovt_hbm, whyt_hbm, wlgt_hbm, bov_hbm, bhy_hbm, blg_hbm,
             oc1_hbm, ogui_hbm,
             s_e0, s_c1, s_wovt, s_whyt, s_wlgt, s_wov, s_why, s_wlg,
             s_bov, s_bhy, s_blg, sem):
        first = ((e0_hbm, s_e0),
                 (wovt_hbm, s_wovt), (whyt_hbm, s_whyt), (wlgt_hbm, s_wlgt),
                 (bov_hbm, s_bov), (bhy_hbm, s_bhy), (blg_hbm, s_blg))
        for src, dst in first:
            _start(src, dst, sem)
        for src, dst in first:
            _wait(src, dst, sem)
        s_wov[...] = s_wovt[...].T
        s_why[...] = s_whyt[...].T
        s_wlg[...] = s_wlgt[...].T

        def lg1_body(lg_ref, o_ref):
            o_ref[...] = jnp.dot(lg_ref[...], s_e0[...],
                                 preferred_element_type=jnp.float32)

        pltpu.emit_pipeline(
            lg1_body,
            grid=(nlg // bm,),
            in_specs=[pl.BlockSpec((bm, nlg), lambda i: (i, 0))],
            out_specs=[pl.BlockSpec((bm, d), lambda i: (i, 0))],
        )(lg_hbm, oc1_hbm)

        _start(oc1_hbm, s_c1, sem)
        _wait(oc1_hbm, s_c1, sem)

        def fuse_body(lg_ref, ge_ref, m1_ref, m2_ref, g0_ref, c1_ref, o_ref):
            c2 = jnp.dot(lg_ref[...], s_c1[...],
                         preferred_element_type=jnp.float32)
            ge_v = ge_ref[...]
            he = ge_v + m1_ref[...] + m2_ref[...]
            lgx = (g0_ref[...] + c1_ref[...] + c2) * (1.0 / 3.0)
            co = jax.nn.sigmoid(
                jnp.dot(ge_v, s_wov[...], preferred_element_type=jnp.float32)
                + s_bov[...])
            ch = jax.nn.sigmoid(
                jnp.dot(he, s_why[...], preferred_element_type=jnp.float32)
                + s_bhy[...])
            cl = jax.nn.sigmoid(
                jnp.dot(lgx, s_wlg[...], preferred_element_type=jnp.float32)
                + s_blg[...])
            v = co * ge_v + ch * he + cl * lgx
            o_ref[...] = jnp.concatenate([v, jnp.zeros_like(v)], axis=1)

        pltpu.emit_pipeline(
            fuse_body,
            grid=(g // bm,),
            in_specs=[pl.BlockSpec((bm, nlg), lambda i: (i, 0)),
                      pl.BlockSpec((bm, d), lambda i: (i, 0)),
                      pl.BlockSpec((bm, d), lambda i: (i, 0)),
                      pl.BlockSpec((bm, d), lambda i: (i, 0)),
                      pl.BlockSpec((bm, d), lambda i: (i, 0)),
                      pl.BlockSpec((bm, d), lambda i: (i, 0))],
            out_specs=[pl.BlockSpec((bm, 2 * d), lambda i: (i, 0))],
        )(lg_hbm, ge_hbm, m1_hbm, m2_hbm, e0_hbm, oc1_hbm, ogui_hbm)

    anyspec = pl.BlockSpec(memory_space=pltpu.MemorySpace.HBM)
    f32 = jnp.float32
    out = pl.pallas_call(
        body,
        in_specs=[anyspec] * 11,
        out_specs=[anyspec] * 2,
        out_shape=(jax.ShapeDtypeStruct((nlg, d), f32),
                   jax.ShapeDtypeStruct((g, 2 * d), f32)),
        scratch_shapes=[pltpu.VMEM((nlg, d), f32),
                        pltpu.VMEM((nlg, d), f32),
                        pltpu.VMEM((1, d), f32),
                        pltpu.VMEM((1, d), f32),
                        pltpu.VMEM((1, d), f32),
                        pltpu.VMEM((d, 1), f32),
                        pltpu.VMEM((d, 1), f32),
                        pltpu.VMEM((d, 1), f32),
                        pltpu.VMEM((1, 1), f32),
                        pltpu.VMEM((1, 1), f32),
                        pltpu.VMEM((1, 1), f32),
                        pltpu.SemaphoreType.DMA],
    )(lg, e0, ge, m1, m2,
      wovt, whyt, wlgt,
      bov.reshape(1, 1), bhy.reshape(1, 1), blg.reshape(1, 1))
    return out[1]


def _sc_gather_pair(gtab, itab, gidx, iidx):
    b = gidx.shape[0]
    d = gtab.shape[1]
    w = 128
    mesh = plsc.VectorSubcoreMesh(core_axis_name="c", subcore_axis_name="s")
    gi2 = gidx.reshape(1, b)
    ii2 = iidx.reshape(1, b)

    @pl.kernel(out_type=(jax.ShapeDtypeStruct((b, d), jnp.float32),
                         jax.ShapeDtypeStruct((b, d), jnp.float32)),
               mesh=mesh,
               scratch_types=[pltpu.SemaphoreType.DMA,
                              pltpu.SemaphoreType.DMA])
    def k(gtab_hbm, itab_hbm, gi_hbm, ii_hbm, og_hbm, oi_hbm, sem1, sem2):
        def body(gi_vmem, ii_vmem, og_vmem, oi_vmem):
            cg = pltpu.make_async_copy(gtab_hbm.at[gi_vmem.at[0]], og_vmem,
                                       sem1)
            ci = pltpu.make_async_copy(itab_hbm.at[ii_vmem.at[0]], oi_vmem,
                                       sem2)
            cg.start()
            ci.start()
            cg.wait()
            ci.wait()

        pltpu.emit_pipeline(
            body,
            grid=(b // w,),
            in_specs=[pl.BlockSpec((1, w), lambda i: (0, i)),
                      pl.BlockSpec((1, w), lambda i: (0, i))],
            out_specs=[pl.BlockSpec((w, d), lambda i: (i, 0)),
                       pl.BlockSpec((w, d), lambda i: (i, 0))],
            core_axis_name=("c", "s"),
            dimension_semantics=(pltpu.PARALLEL,),
        )(gi_hbm, ii_hbm, og_hbm, oi_hbm)

    return k(gtab, itab, gi2, ii2)


def _dot_body(g_ref, i_ref, o_ref):
    s = jnp.sum(g_ref[...] * i_ref[...], axis=1)
    o_ref[...] = s.reshape(1, s.shape[0])


def _dot(gs, isel, bm):
    b, d = gs.shape
    out = pl.pallas_call(
        _dot_body,
        grid=(b // bm,),
        in_specs=[pl.BlockSpec((bm, d), lambda i: (i, 0)),
                  pl.BlockSpec((bm, d), lambda i: (i, 0))],
        out_specs=pl.BlockSpec((1, bm), lambda i: (0, i)),
        out_shape=jax.ShapeDtypeStruct((1, b), jnp.float32),
    )(gs, isel)
    return out.reshape(b)


def kernel(user_table, item_table, group_table, user_hyper, item_hyper,
           full_hyper, overlap_graph, lgcn_graph, W_agg, b_agg,
           W_ov, b_ov, W_hy, b_hy, W_lg, b_lg,
           group_inputs, item_inputs):
    nlg = lgcn_graph.shape[0]

    ge, m1, m2, i_emb, e0 = _mk1(
        user_table.T, item_table.T, group_table.T, user_hyper, item_hyper,
        full_hyper.T, overlap_graph, W_agg.transpose(0, 2, 1), b_agg, nlg)

    group_ui = _mk2(lgcn_graph, e0, ge, m1, m2,
                    W_ov.T, W_hy.T, W_lg.T, b_ov, b_hy, b_lg)

    g_sel, i_sel = _sc_gather_pair(group_ui, i_emb, group_inputs, item_inputs)
    return _dot(g_sel, i_sel, bm=4096)
